# Initial kernel scaffold; baseline (speedup 1.0000x reference)
import jax
import jax.numpy as jnp
from jax.experimental import pallas as pl


def kernel(x, edge_index, edge_weight, W1, b1, Wl, bl):
    raise NotImplementedError("write your pallas kernel here")


# speedup vs baseline: 6.2323x; 6.2323x over previous
import functools

import jax
import jax.numpy as jnp
from jax import lax
from jax.experimental import pallas as pl
from jax.experimental.pallas import tpu as pltpu
from jax.experimental.pallas import tpu_sc as plsc

N = 10000
E = 160000
F = 256
H = 512
O = 256

NC = 2
NS = 16
L = 16
E_PAD = 163840
EPT = E_PAD // NS
HALF = N // 2
QUARTER = HALF // 2
QPAD = 2560
ROWS_PT = QPAD // NS
BOUNCE = 40
K = 16


def _fast_rsqrt(d):
    i = lax.bitcast_convert_type(d, jnp.int32)
    i = jnp.int32(0x5F3759DF) - (i >> 1)
    y = lax.bitcast_convert_type(i, jnp.float32)
    for _ in range(3):
        y = y * (1.5 - 0.5 * d * y * y)
    return y


def _sc_body(x_hbm, row_hbm, col_hbm, ew_hbm,
             agg_hbm, dinv_hbm,
             rowv, colv, eww, dinv, rlist, clist, slist,
             gbuf, bounce, sh_deg, sh_agg, sem):
    c = lax.axis_index("c")
    s = lax.axis_index("s")
    base_e = s * EPT

    pltpu.sync_copy(row_hbm.at[pl.ds(base_e, EPT)], rowv)
    pltpu.sync_copy(col_hbm.at[pl.ds(base_e, EPT)], colv)
    pltpu.sync_copy(ew_hbm.at[pl.ds(base_e, EPT)], eww)

    zv = jnp.zeros((L,), jnp.float32)
    ziv = jnp.zeros((L,), jnp.int32)

    def zero_bounce(r, _):
        for ffi in range(F // L):
            bounce[r, pl.ds(ffi * L, L)] = zv
        return 0

    def zero_deg(j, _):
        dinv[pl.ds(j * L, L)] = zv
        return 0
    lax.fori_loop(0, EPT // L, zero_deg, 0)
    pltpu.sync_copy(dinv.at[pl.ds(0, EPT // NS)],
                    sh_deg.at[pl.ds(s * (EPT // NS), EPT // NS)])
    plsc.subcore_barrier()

    def deg_step(j, _):
        cv = colv[pl.ds(j * L, L)]
        pltpu.sync_copy(eww.at[pl.ds(j * L, L)], sh_deg.at[cv], add=True)
        return 0
    lax.fori_loop(0, EPT // L, deg_step, 0)
    plsc.subcore_barrier()

    pltpu.sync_copy(sh_deg, dinv)

    def dinv_step(j, _):
        d = dinv[pl.ds(j * L, L)]
        dinv[pl.ds(j * L, L)] = _fast_rsqrt(d + 1.0)
        return 0
    lax.fori_loop(0, EPT // L, dinv_step, 0)

    for q in range(2):
        nbase = (2 * c + q) * QUARTER

        lax.fori_loop(0, BOUNCE, zero_bounce, 0)
        for b in range(ROWS_PT // BOUNCE):
            pltpu.sync_copy(bounce,
                            sh_agg.at[pl.ds(s * ROWS_PT + b * BOUNCE, BOUNCE), :])

        def zero_lists(j, _):
            rlist[pl.ds(j * L, L)] = ziv
            clist[pl.ds(j * L, L)] = ziv
            slist[pl.ds(j * L, L)] = zv
            return 0
        lax.fori_loop(0, EPT // L, zero_lists, 0)

        def compact_step(j, cnt):
            rv = rowv[pl.ds(j * L, L)]
            cv = colv[pl.ds(j * L, L)]
            wv = eww[pl.ds(j * L, L)]
            m = (cv >= nbase) & (cv < nbase + QUARTER)
            sv = wv * plsc.load_gather(dinv, [rv])
            plsc.store_compressed(rlist.at[pl.ds(cnt, L)], rv, mask=m)
            plsc.store_compressed(clist.at[pl.ds(cnt, L)], cv - nbase, mask=m)
            plsc.store_compressed(slist.at[pl.ds(cnt, L)], sv, mask=m)
            return cnt + jnp.sum(m.astype(jnp.int32))
        cnt = lax.fori_loop(0, EPT // L, compact_step, jnp.int32(0))
        rlist[pl.ds(cnt, L)] = ziv
        clist[pl.ds(cnt, L)] = ziv
        slist[pl.ds(cnt, L)] = zv
        plsc.subcore_barrier()

        nch = (cnt + (K - 1)) // K

        def agg_step(j, _):
            rv = rlist[pl.ds(j * K, K)]
            pltpu.async_copy(x_hbm.at[rv], gbuf, sem).wait()
            svc = slist[pl.ds(j * K, K)]
            for rr in range(K):
                sp = jnp.full((L,), svc[rr])
                for ff in range(F // L):
                    gbuf[rr, pl.ds(ff * L, L)] = \
                        gbuf[rr, pl.ds(ff * L, L)] * sp
            cv = clist[pl.ds(j * K, K)]
            pltpu.sync_copy(gbuf, sh_agg.at[cv], add=True)
            return 0
        lax.fori_loop(0, nch, agg_step, 0)
        plsc.subcore_barrier()

        for b in range(ROWS_PT // BOUNCE):
            pltpu.sync_copy(sh_agg.at[pl.ds(s * ROWS_PT + b * BOUNCE, BOUNCE), :],
                            bounce)
            pltpu.sync_copy(bounce,
                            agg_hbm.at[2 * c + q,
                                       pl.ds(s * ROWS_PT + b * BOUNCE, BOUNCE), :])

    @pl.when((c == 0) & (s == 0))
    def _():
        pltpu.sync_copy(dinv, dinv_hbm)


def _sc_propagate(x, rowp, colp, ewp):
    mesh = plsc.VectorSubcoreMesh(core_axis_name="c", subcore_axis_name="s")
    kfn = pl.kernel(
        _sc_body,
        out_type=(
            jax.ShapeDtypeStruct((4, QPAD, F), jnp.float32),
            jax.ShapeDtypeStruct((EPT,), jnp.float32),
        ),
        mesh=mesh,
        scratch_types=[
            pltpu.VMEM((EPT,), jnp.int32),
            pltpu.VMEM((EPT,), jnp.int32),
            pltpu.VMEM((EPT,), jnp.float32),
            pltpu.VMEM((EPT,), jnp.float32),
            pltpu.VMEM((EPT + L,), jnp.int32),
            pltpu.VMEM((EPT + L,), jnp.int32),
            pltpu.VMEM((EPT + L,), jnp.float32),
            pltpu.VMEM((K, F), jnp.float32),
            pltpu.VMEM((BOUNCE, F), jnp.float32),
            pltpu.VMEM_SHARED((EPT,), jnp.float32),
            pltpu.VMEM_SHARED((QPAD, F), jnp.float32),
            pltpu.SemaphoreType.DMA,
        ],
        compiler_params=pltpu.CompilerParams(use_tc_tiling_on_sc=False,
                                             needs_layout_passes=False),
    )
    return kfn(x, rowp, colp, ewp)


def kernel(x, edge_index, edge_weight, W1, b1, Wl, bl):
    row = edge_index[0]
    col = edge_index[1]
    pad = E_PAD - E
    rowp = jnp.concatenate([row, jnp.zeros((pad,), row.dtype)])
    colp = jnp.concatenate([col, jnp.zeros((pad,), col.dtype)])
    ewp = jnp.concatenate([edge_weight, jnp.zeros((pad,), edge_weight.dtype)])
    aggq, dinv = _sc_propagate(x, rowp, colp, ewp)
    agg = jnp.concatenate([aggq[0, :QUARTER], aggq[1, :QUARTER],
                           aggq[2, :QUARTER], aggq[3, :QUARTER]], axis=0)
    dinv2d = dinv[:N].reshape(N, 1)
    return _tc_dense2(x, agg, dinv2d, W1, b1.reshape(1, H),
                      Wl, bl.reshape(1, O))


NB = 2000


def _tc_body2(x_ref, agg_ref, dinv_ref, w1_ref, b1_ref, wl_ref, bl_ref,
              o_ref):
    dv = dinv_ref[...]
    hpre = dv * agg_ref[...] + (dv * dv) * x_ref[...]
    h = jnp.dot(hpre, w1_ref[...], preferred_element_type=jnp.float32)
    h = jnp.maximum(h + b1_ref[...], 0.0)
    o = jnp.dot(h, wl_ref[...], preferred_element_type=jnp.float32)
    o_ref[...] = jnp.maximum(o + bl_ref[...], 0.0)


def _tc_dense2(x, agg, dinv2d, W1, b1, Wl, bl):
    return pl.pallas_call(
        _tc_body2,
        grid=(N // NB,),
        in_specs=[
            pl.BlockSpec((NB, F), lambda i: (i, 0)),
            pl.BlockSpec((NB, F), lambda i: (i, 0)),
            pl.BlockSpec((NB, 1), lambda i: (i, 0)),
            pl.BlockSpec((F, H), lambda i: (0, 0)),
            pl.BlockSpec((1, H), lambda i: (0, 0)),
            pl.BlockSpec((H, O), lambda i: (0, 0)),
            pl.BlockSpec((1, O), lambda i: (0, 0)),
        ],
        out_specs=pl.BlockSpec((NB, O), lambda i: (i, 0)),
        out_shape=jax.ShapeDtypeStruct((N, O), jnp.float32),
    )(x, agg, dinv2d, W1, b1, Wl, bl)


# speedup vs baseline: 8.5916x; 1.3786x over previous
import functools

import jax
import jax.numpy as jnp
from jax import lax
from jax.experimental import pallas as pl
from jax.experimental.pallas import tpu as pltpu
from jax.experimental.pallas import tpu_sc as plsc

N = 10000
E = 160000
F = 256
H = 512
O = 256

NC = 2
NS = 16
L = 16
E_PAD = 163840
EPT = E_PAD // NS
HALF = N // 2
QUARTER = HALF // 2
QPAD = 2560
ROWS_PT = QPAD // NS
BOUNCE = 8
K = 16


def _fast_rsqrt(d):
    i = lax.bitcast_convert_type(d, jnp.int32)
    i = jnp.int32(0x5F3759DF) - (i >> 1)
    y = lax.bitcast_convert_type(i, jnp.float32)
    for _ in range(3):
        y = y * (1.5 - 0.5 * d * y * y)
    return y


def _sc_body(x_hbm, row_hbm, col_hbm, ew_hbm,
             agg_hbm, dinv_hbm,
             rowv, colv, eww, dinv, rlist, clist, slist,
             gbuf, gbuf2, bounce, sh_deg, sh_agg, sem, sem2):
    c = lax.axis_index("c")
    s = lax.axis_index("s")
    base_e = s * EPT

    pltpu.sync_copy(row_hbm.at[pl.ds(base_e, EPT)], rowv)
    pltpu.sync_copy(col_hbm.at[pl.ds(base_e, EPT)], colv)
    pltpu.sync_copy(ew_hbm.at[pl.ds(base_e, EPT)], eww)

    zv = jnp.zeros((L,), jnp.float32)
    ziv = jnp.zeros((L,), jnp.int32)

    def zero_bounce(r, _):
        for ffi in range(F // L):
            bounce[r, pl.ds(ffi * L, L)] = zv
        return 0

    def zero_deg(j, _):
        dinv[pl.ds(j * L, L)] = zv
        return 0
    lax.fori_loop(0, EPT // L, zero_deg, 0)
    pltpu.sync_copy(dinv.at[pl.ds(0, EPT // NS)],
                    sh_deg.at[pl.ds(s * (EPT // NS), EPT // NS)])
    plsc.subcore_barrier()

    def deg_step(j, _):
        cv = colv[pl.ds(j * L, L)]
        pltpu.sync_copy(eww.at[pl.ds(j * L, L)], sh_deg.at[cv], add=True)
        return 0
    lax.fori_loop(0, EPT // L, deg_step, 0)
    plsc.subcore_barrier()

    pltpu.sync_copy(sh_deg, dinv)

    def dinv_step(j, _):
        d = dinv[pl.ds(j * L, L)]
        dinv[pl.ds(j * L, L)] = _fast_rsqrt(d + 1.0)
        return 0
    lax.fori_loop(0, EPT // L, dinv_step, 0)

    for q in range(2):
        nbase = (2 * c + q) * QUARTER

        lax.fori_loop(0, BOUNCE, zero_bounce, 0)
        for b in range(ROWS_PT // BOUNCE):
            pltpu.sync_copy(bounce,
                            sh_agg.at[pl.ds(s * ROWS_PT + b * BOUNCE, BOUNCE), :])

        def zero_lists(j, _):
            rlist[pl.ds(j * L, L)] = ziv
            clist[pl.ds(j * L, L)] = ziv
            slist[pl.ds(j * L, L)] = zv
            return 0
        lax.fori_loop(0, EPT // L, zero_lists, 0)

        def compact_step(j, cnt):
            rv = rowv[pl.ds(j * L, L)]
            cv = colv[pl.ds(j * L, L)]
            wv = eww[pl.ds(j * L, L)]
            m = (cv >= nbase) & (cv < nbase + QUARTER)
            sv = wv * plsc.load_gather(dinv, [rv])
            plsc.store_compressed(rlist.at[pl.ds(cnt, L)], rv, mask=m)
            plsc.store_compressed(clist.at[pl.ds(cnt, L)], cv - nbase, mask=m)
            plsc.store_compressed(slist.at[pl.ds(cnt, L)], sv, mask=m)
            return cnt + jnp.sum(m.astype(jnp.int32))
        cnt = lax.fori_loop(0, EPT // L, compact_step, jnp.int32(0))
        rlist[pl.ds(cnt, L)] = ziv
        clist[pl.ds(cnt, L)] = ziv
        slist[pl.ds(cnt, L)] = zv
        plsc.subcore_barrier()

        nch = (cnt + (K - 1)) // K

        def start_gather(j, buf, sm):
            pltpu.async_copy(x_hbm.at[rlist[pl.ds(j * K, K)]], buf, sm)

        def wait_gather(buf, sm):
            pltpu.make_async_copy(x_hbm.at[pl.ds(0, K)], buf, sm).wait()

        def scale_scatter(j, buf):
            svc = slist[pl.ds(j * K, K)]
            for rr in range(K):
                sp = jnp.full((L,), svc[rr])
                for ff in range(F // L):
                    buf[rr, pl.ds(ff * L, L)] = \
                        buf[rr, pl.ds(ff * L, L)] * sp
            cv = clist[pl.ds(j * K, K)]
            pltpu.sync_copy(buf, sh_agg.at[cv], add=True)

        @pl.when(nch > 0)
        def _():
            start_gather(0, gbuf, sem)

        def agg2_step(jj, _):
            j0 = 2 * jj
            j1 = j0 + 1

            @pl.when(j0 < nch)
            def _():
                wait_gather(gbuf, sem)

                @pl.when(j1 < nch)
                def _():
                    start_gather(j1, gbuf2, sem2)
                scale_scatter(j0, gbuf)

                @pl.when(j0 + 2 < nch)
                def _():
                    start_gather(j0 + 2, gbuf, sem)

            @pl.when(j1 < nch)
            def _():
                wait_gather(gbuf2, sem2)
                scale_scatter(j1, gbuf2)
            return 0
        lax.fori_loop(0, (nch + 1) // 2, agg2_step, 0)
        plsc.subcore_barrier()

        for b in range(ROWS_PT // BOUNCE):
            pltpu.sync_copy(sh_agg.at[pl.ds(s * ROWS_PT + b * BOUNCE, BOUNCE), :],
                            bounce)
            pltpu.sync_copy(bounce,
                            agg_hbm.at[2 * c + q,
                                       pl.ds(s * ROWS_PT + b * BOUNCE, BOUNCE), :])

    @pl.when((c == 0) & (s == 0))
    def _():
        pltpu.sync_copy(dinv, dinv_hbm)


def _sc_propagate(x, rowp, colp, ewp):
    mesh = plsc.VectorSubcoreMesh(core_axis_name="c", subcore_axis_name="s")
    kfn = pl.kernel(
        _sc_body,
        out_type=(
            jax.ShapeDtypeStruct((4, QPAD, F), jnp.float32),
            jax.ShapeDtypeStruct((EPT,), jnp.float32),
        ),
        mesh=mesh,
        scratch_types=[
            pltpu.VMEM((EPT,), jnp.int32),
            pltpu.VMEM((EPT,), jnp.int32),
            pltpu.VMEM((EPT,), jnp.float32),
            pltpu.VMEM((EPT,), jnp.float32),
            pltpu.VMEM((EPT + L,), jnp.int32),
            pltpu.VMEM((EPT + L,), jnp.int32),
            pltpu.VMEM((EPT + L,), jnp.float32),
            pltpu.VMEM((K, F), jnp.float32),
            pltpu.VMEM((K, F), jnp.float32),
            pltpu.VMEM((BOUNCE, F), jnp.float32),
            pltpu.VMEM_SHARED((EPT,), jnp.float32),
            pltpu.VMEM_SHARED((QPAD, F), jnp.float32),
            pltpu.SemaphoreType.DMA,
            pltpu.SemaphoreType.DMA,
        ],
        compiler_params=pltpu.CompilerParams(use_tc_tiling_on_sc=False,
                                             needs_layout_passes=False),
    )
    return kfn(x, rowp, colp, ewp)


def kernel(x, edge_index, edge_weight, W1, b1, Wl, bl):
    row = edge_index[0]
    col = edge_index[1]
    pad = E_PAD - E
    rowp = jnp.concatenate([row, jnp.zeros((pad,), row.dtype)])
    colp = jnp.concatenate([col, jnp.zeros((pad,), col.dtype)])
    ewp = jnp.concatenate([edge_weight, jnp.zeros((pad,), edge_weight.dtype)])
    aggq, dinv = _sc_propagate(x, rowp, colp, ewp)
    agg = jnp.concatenate([aggq[0, :QUARTER], aggq[1, :QUARTER],
                           aggq[2, :QUARTER], aggq[3, :QUARTER]], axis=0)
    dinv2d = dinv[:N].reshape(N, 1)
    return _tc_dense2(x, agg, dinv2d, W1, b1.reshape(1, H),
                      Wl, bl.reshape(1, O))


NB = 2000


def _tc_body2(x_ref, agg_ref, dinv_ref, w1_ref, b1_ref, wl_ref, bl_ref,
              o_ref):
    dv = dinv_ref[...]
    hpre = dv * agg_ref[...] + (dv * dv) * x_ref[...]
    h = jnp.dot(hpre, w1_ref[...], preferred_element_type=jnp.float32)
    h = jnp.maximum(h + b1_ref[...], 0.0)
    o = jnp.dot(h, wl_ref[...], preferred_element_type=jnp.float32)
    o_ref[...] = jnp.maximum(o + bl_ref[...], 0.0)


def _tc_dense2(x, agg, dinv2d, W1, b1, Wl, bl):
    return pl.pallas_call(
        _tc_body2,
        grid=(N // NB,),
        in_specs=[
            pl.BlockSpec((NB, F), lambda i: (i, 0)),
            pl.BlockSpec((NB, F), lambda i: (i, 0)),
            pl.BlockSpec((NB, 1), lambda i: (i, 0)),
            pl.BlockSpec((F, H), lambda i: (0, 0)),
            pl.BlockSpec((1, H), lambda i: (0, 0)),
            pl.BlockSpec((H, O), lambda i: (0, 0)),
            pl.BlockSpec((1, O), lambda i: (0, 0)),
        ],
        out_specs=pl.BlockSpec((NB, O), lambda i: (i, 0)),
        out_shape=jax.ShapeDtypeStruct((N, O), jnp.float32),
    )(x, agg, dinv2d, W1, b1, Wl, bl)
